# baseline (speedup 1.0000x reference)
import numpy as np
import jax
import jax.numpy as jnp
from jax import lax
from jax.experimental import pallas as pl
from jax.experimental.pallas import tpu as pltpu
from jax.experimental.pallas import tpu_sc as plsc

MAX_LEN = 200
EMBED_DIM = 128
BATCH = 4096

NUM_CORES = 2
NUM_SUBCORES = 16
NUM_WORKERS = NUM_CORES * NUM_SUBCORES

HALF = MAX_LEN // 2
NITEMS = BATCH * 2
IPW = NITEMS // NUM_WORKERS
NBUF = 8
GAHEAD = 4
LANES = 16
DCHUNKS = EMBED_DIM // LANES


def _pe_np():
    pos = np.arange(MAX_LEN, dtype=np.float64)[:, None]
    j = np.arange(EMBED_DIM, dtype=np.float64)[None, :]
    angle = pos / (10000.0 ** (j / float(EMBED_DIM)))
    even = (np.arange(EMBED_DIM)[None, :] % 2) == 0
    return np.where(even, np.sin(angle), np.cos(angle)).astype(np.float32)


_PE = _pe_np()


def _body(x_hbm, pe_hbm, tab_hbm, out_hbm, pe_v, *refs):
    bufs = refs[:NBUF]
    ibufs = refs[NBUF:2 * NBUF]
    gsems = refs[2 * NBUF:3 * NBUF]
    ssems = refs[3 * NBUF:4 * NBUF]
    isems = refs[4 * NBUF:5 * NBUF]

    wid = lax.axis_index("s") * NUM_CORES + lax.axis_index("c")
    item0 = wid * IPW

    pltpu.sync_copy(pe_hbm, pe_v)

    def fire_idx(k, p):
        pltpu.async_copy(x_hbm.at[item0 + k], ibufs[p], isems[p])

    def drain_idx(p):
        pltpu.make_async_copy(x_hbm.at[0], ibufs[p], isems[p]).wait()

    def fire_gather(k, p):
        drain_idx(p)
        pltpu.async_copy(tab_hbm.at[ibufs[p]], bufs[p], gsems[p])

    def drain_gather(p):
        pltpu.make_async_copy(out_hbm.at[0], bufs[p], gsems[p]).wait()

    def fire_store(k, p):
        pltpu.async_copy(bufs[p], out_hbm.at[item0 + k], ssems[p])

    def drain_store(p):
        pltpu.make_async_copy(bufs[p], out_hbm.at[0], ssems[p]).wait()

    def add_pe(p):
        poff = (p % 2) * HALF

        def t_body(t, carry):
            for d in range(DCHUNKS):
                sl = pl.ds(LANES * d, LANES)
                plsc.addupdate(bufs[p].at[t, sl], pe_v[poff + t, sl])
            return carry
        lax.fori_loop(0, HALF, t_body, 0, unroll=4)

    for q in range(NBUF - 1):
        fire_idx(q, q)
    for q in range(GAHEAD):
        fire_gather(q, q)

    def j_body(j, carry):
        for p in range(NBUF):
            k = NBUF * j + p
            drain_gather(p)
            add_pe(p)
            fire_store(k, p)

            @pl.when(k + NBUF - 1 < IPW)
            def _():
                fire_idx(k + NBUF - 1, (p + NBUF - 1) % NBUF)

            @pl.when(k + GAHEAD < IPW)
            def _():
                @pl.when(k >= NBUF - GAHEAD)
                def _():
                    drain_store((p + GAHEAD) % NBUF)
                fire_gather(k + GAHEAD, (p + GAHEAD) % NBUF)
        return carry

    lax.fori_loop(0, IPW // NBUF, j_body, 0)

    for p in range(NBUF):
        drain_store(p)


_run = pl.kernel(
    _body,
    out_type=jax.ShapeDtypeStruct((NITEMS, HALF, EMBED_DIM), jnp.float32),
    mesh=plsc.VectorSubcoreMesh(core_axis_name="c", subcore_axis_name="s"),
    compiler_params=pltpu.CompilerParams(use_tc_tiling_on_sc=False),
    scratch_types=(
        [pltpu.VMEM((MAX_LEN, EMBED_DIM), jnp.float32)]
        + [pltpu.VMEM((HALF, EMBED_DIM), jnp.float32)] * NBUF
        + [pltpu.VMEM((HALF,), jnp.int32)] * NBUF
        + [pltpu.SemaphoreType.DMA] * (3 * NBUF)
    ),
)


def kernel(x, embed_weight):
    x2 = x.astype(jnp.int32).reshape(NITEMS, HALF)
    pe = jnp.asarray(_PE)
    out = _run(x2, pe, embed_weight)
    return out.reshape(BATCH, MAX_LEN, EMBED_DIM)
